# baseline (speedup 1.0000x reference)
import functools

import jax
import jax.numpy as jnp
from jax import lax
from jax.experimental import pallas as pl
from jax.experimental.pallas import tpu as pltpu
from jax.experimental.pallas import tpu_sc as plsc

_B, _S, _D, _H, _E, _C = 2, 2048, 1024, 2048, 4, 1000
_N = _B * _S
_EP = 128
_RB = 512
_NRB = _N // _RB
_TB = 256
_NBLK = _N // _TB
_NWORK = _NBLK + _E - 1
_SB = 256
_NSB = _S // _SB

_NC = 2
_NS = 16
_NW = _NC * _NS
_RPW = _N // _NW
_GCH = 32
_NCHK = _RPW // _GCH


def _router_body(x_ref, w_ref, b_ref, eid_ref, gbw_ref, rank_ref, cnt_ref,
                 aux_ref, acc_ref):
    i = pl.program_id(0)

    @pl.when(i == 0)
    def _():
        acc_ref[...] = jnp.zeros_like(acc_ref)

    xb = x_ref[...]
    logits = jnp.dot(xb, w_ref[...], preferred_element_type=jnp.float32)
    logits = logits + b_ref[...]
    m = jnp.max(logits, axis=-1, keepdims=True)
    p = jnp.exp(logits - m)
    s = jnp.sum(p, axis=-1, keepdims=True)
    probs = p / s
    eid = jnp.argmax(logits, axis=-1).astype(jnp.int32)
    gate = 1.0 / s[:, 0]
    eid_ref[0, 0, :] = eid
    row = lax.broadcasted_iota(jnp.int32, (_RB, 1), 0) + i * _RB
    bw = (row < _S).astype(jnp.float32)
    lanes = lax.broadcasted_iota(jnp.int32, (_RB, 128), 1)
    gbw_ref[...] = jnp.where(lanes == 0, gate[:, None],
                             jnp.where(lanes == 1, bw, 0.0))
    onehot = (lax.broadcasted_iota(jnp.int32, (_RB, _EP), 1)
              == eid[:, None]).astype(jnp.float32)
    tri = (lax.broadcasted_iota(jnp.int32, (_RB, _RB), 0)
           > lax.broadcasted_iota(jnp.int32, (_RB, _RB), 1)).astype(jnp.float32)
    prefix = jnp.dot(tri, onehot, preferred_element_type=jnp.float32)
    rank = jnp.sum(onehot * (acc_ref[1:2, :] + prefix), axis=1)
    rank_ref[0, 0, :] = rank.astype(jnp.int32)
    acc_ref[0:1, :] += jnp.sum(probs, axis=0, keepdims=True)
    acc_ref[1:2, :] += jnp.sum(onehot, axis=0, keepdims=True)

    @pl.when(i == _NRB - 1)
    def _():
        cnt_ref[...] = acc_ref[1:2, :]
        aux = (_E / (_N * _N)) * jnp.sum(acc_ref[0:1, :] * acc_ref[1:2, :])
        aux_ref[...] = aux * jnp.ones((1, _EP), jnp.float32)


def _run_router(x_flat, router_w, router_b):
    wp = jnp.pad(router_w, ((0, 0), (0, _EP - _E)))
    bp = jnp.full((1, _EP), -jnp.inf, jnp.float32).at[0, :_E].set(router_b)
    eid3, gbw, rank3, cnt, aux = pl.pallas_call(
        _router_body,
        grid=(_NRB,),
        in_specs=[
            pl.BlockSpec((_RB, _D), lambda i: (i, 0)),
            pl.BlockSpec((_D, _EP), lambda i: (0, 0)),
            pl.BlockSpec((1, _EP), lambda i: (0, 0)),
        ],
        out_specs=[
            pl.BlockSpec((1, 1, _RB), lambda i: (i, 0, 0)),
            pl.BlockSpec((_RB, 128), lambda i: (i, 0)),
            pl.BlockSpec((1, 1, _RB), lambda i: (i, 0, 0)),
            pl.BlockSpec((1, _EP), lambda i: (0, 0)),
            pl.BlockSpec((1, _EP), lambda i: (0, 0)),
        ],
        out_shape=[
            jax.ShapeDtypeStruct((_NRB, 1, _RB), jnp.int32),
            jax.ShapeDtypeStruct((_N, 128), jnp.float32),
            jax.ShapeDtypeStruct((_NRB, 1, _RB), jnp.int32),
            jax.ShapeDtypeStruct((1, _EP), jnp.float32),
            jax.ShapeDtypeStruct((1, _EP), jnp.float32),
        ],
        scratch_shapes=[pltpu.VMEM((8, _EP), jnp.float32)],
    )(x_flat, wp, bp)
    return eid3, gbw, rank3, cnt, aux[0, 0]


def _dst_body(eid_ref, rank_ref, cnt_ref, dst_ref, off_ref):
    c0 = cnt_ref[0, 0].astype(jnp.int32)
    c1 = cnt_ref[0, 1].astype(jnp.int32)
    c2 = cnt_ref[0, 2].astype(jnp.int32)
    o1 = c0
    o2 = c0 + c1
    o3 = o2 + c2
    eidb = eid_ref[...]
    sel = jnp.where(eidb == 0, 0,
                    jnp.where(eidb == 1, o1,
                              jnp.where(eidb == 2, o2, o3)))
    dst_ref[...] = sel + rank_ref[...]
    lane = lax.broadcasted_iota(jnp.int32, (1, _EP), 1)
    off_ref[...] = (jnp.where(lane == 1, o1,
                              jnp.where(lane == 2, o2,
                                        jnp.where(lane == 3, o3,
                                                  jnp.where(lane == 4, _N,
                                                            0)))))


def _run_dst(eid3, rank3, cnt):
    return pl.pallas_call(
        _dst_body,
        out_shape=[
            jax.ShapeDtypeStruct((_NRB, 1, _RB), jnp.int32),
            jax.ShapeDtypeStruct((1, _EP), jnp.int32),
        ],
    )(eid3, rank3, cnt)


_HC = 512
_NHC = _H // _HC


def _ffn_body(off_ref, xs_ref, w1_ref, b1_ref, w2_ref, b2_ref, gbw_ref,
              lw_ref, lb_ref, out_ref, logits_ref, pacc_ref):
    e = pl.program_id(0)
    hc = pl.program_id(1)

    @pl.when((e == 0) & (hc == 0))
    def _():
        pacc_ref[...] = jnp.zeros_like(pacc_ref)

    lo = off_ref[e]
    hi = off_ref[e + 1]

    @pl.when(hi > lo)
    def _():
        sblk = lo // _TB
        eblk = (hi - 1) // _TB
        w1c = w1_ref[0]
        w2c = w2_ref[0]
        b1c = b1_ref[0, 0]
        b2c = b2_ref[0, 0]

        def blk(i, carry):
            b = sblk + i
            r0 = b * _TB
            xb = xs_ref[pl.ds(r0, _TB), :]
            hh = jnp.dot(xb, w1c, preferred_element_type=jnp.float32)
            hh = jnp.maximum(hh + b1c, 0.0)
            yc = jnp.dot(hh, w2c, preferred_element_type=jnp.float32)
            ri = lax.broadcasted_iota(jnp.int32, (1, _TB), 1)[0] + r0
            sel = (ri >= lo) & (ri < hi)
            gb = gbw_ref[pl.ds(r0, _TB), :]
            wt = jnp.where(sel, gb[:, 0], 0.0)
            bias_on = jnp.where(hc == 0, 1.0, 0.0)
            contrib = (yc + b2c * bias_on) * wt[:, None]
            @pl.when((hc == 0) & (lo <= r0))
            def _():
                out_ref[pl.ds(r0, _TB), :] = contrib

            @pl.when((hc != 0) | (lo > r0))
            def _():
                out_ref[pl.ds(r0, _TB), :] += contrib
            s_all = jnp.sum(contrib, axis=0, keepdims=True)
            s_b0 = jnp.sum(contrib * gb[:, 1][:, None], axis=0,
                           keepdims=True)
            pacc_ref[0:1, :] += s_b0
            pacc_ref[1:2, :] += s_all - s_b0
            return carry

        lax.fori_loop(0, eblk - sblk + 1, blk, 0)

    @pl.when((e == _E - 1) & (hc == _NHC - 1))
    def _():
        pooled = pacc_ref[0:_B, :] * (1.0 / _S)
        logits_ref[...] = (jnp.dot(pooled, lw_ref[...],
                                   preferred_element_type=jnp.float32)
                           + lb_ref[...])


def _run_ffn(xs, W1, b1, W2, b2, gbw_s, lin_w, lin_b, off):
    grid_spec = pltpu.PrefetchScalarGridSpec(
        num_scalar_prefetch=1,
        grid=(_E, _NHC),
        in_specs=[
            pl.BlockSpec((_N, _D), lambda e, hc, off: (0, 0)),
            pl.BlockSpec((1, _D, _HC), lambda e, hc, off: (e, 0, hc)),
            pl.BlockSpec((1, 1, _HC), lambda e, hc, off: (e, 0, hc)),
            pl.BlockSpec((1, _HC, _D), lambda e, hc, off: (e, hc, 0)),
            pl.BlockSpec((1, 1, _D), lambda e, hc, off: (e, 0, 0)),
            pl.BlockSpec((_N, 128), lambda e, hc, off: (0, 0)),
            pl.BlockSpec((_D, _C), lambda e, hc, off: (0, 0)),
            pl.BlockSpec((1, _C), lambda e, hc, off: (0, 0)),
        ],
        out_specs=[
            pl.BlockSpec((_N, _D), lambda e, hc, off: (0, 0)),
            pl.BlockSpec((_B, _C), lambda e, hc, off: (0, 0)),
        ],
        scratch_shapes=[pltpu.VMEM((8, _D), jnp.float32)],
    )
    return pl.pallas_call(
        _ffn_body,
        grid_spec=grid_spec,
        out_shape=[
            jax.ShapeDtypeStruct((_N, _D), jnp.float32),
            jax.ShapeDtypeStruct((_B, _C), jnp.float32),
        ],
    )(off, xs, W1, b1.reshape(_E, 1, _H), W2, b2.reshape(_E, 1, _D), gbw_s,
      lin_w, lin_b.reshape(1, _C))


def _sc_gather_body(table_hbm, idx_hbm, out_hbm, idx_v, rows_v, sem0, sem1):
    wid = lax.axis_index("s") * _NC + lax.axis_index("c")
    base = wid * _RPW
    pltpu.sync_copy(idx_hbm.at[pl.ds(base, _RPW)], idx_v)
    sems = (sem0, sem1)
    handles = [None, None]
    for j in range(_NCHK):
        handles[j % 2] = pltpu.async_copy(
            table_hbm.at[idx_v.at[pl.ds(j * _GCH, _GCH)]], rows_v.at[j % 2],
            sems[j % 2])
        if j > 0:
            handles[(j - 1) % 2].wait()
            pltpu.sync_copy(rows_v.at[(j - 1) % 2],
                            out_hbm.at[pl.ds(base + (j - 1) * _GCH, _GCH)])
    handles[(_NCHK - 1) % 2].wait()
    pltpu.sync_copy(rows_v.at[(_NCHK - 1) % 2],
                    out_hbm.at[pl.ds(base + (_NCHK - 1) * _GCH, _GCH)])


def _sc_gather_rows(table, idx):
    k = functools.partial(
        pl.kernel,
        mesh=plsc.VectorSubcoreMesh(core_axis_name="c", subcore_axis_name="s",
                                    num_cores=_NC, num_subcores=_NS),
        out_type=jax.ShapeDtypeStruct((_N, _D), jnp.float32),
        scratch_types=[
            pltpu.VMEM((_RPW,), jnp.int32),
            pltpu.VMEM((2, _GCH, _D), jnp.float32),
            pltpu.SemaphoreType.DMA,
            pltpu.SemaphoreType.DMA,
        ],
    )(_sc_gather_body)
    return k(table, idx)


def _sc_scatter_body(src_hbm, gbw_hbm, idx_hbm, out_hbm, gbws_hbm, idx_v,
                     rows_v, grows_v, sem0, sem1, gsem):
    wid = lax.axis_index("s") * _NC + lax.axis_index("c")
    base = wid * _RPW
    pltpu.sync_copy(idx_hbm.at[wid], idx_v)
    sems = (sem0, sem1)
    handles = [None, None]
    for j in range(_NCHK):
        if j >= 2:
            handles[j % 2].wait()
        pltpu.sync_copy(src_hbm.at[pl.ds(base + j * _GCH, _GCH)],
                        rows_v.at[j % 2])
        handles[j % 2] = pltpu.async_copy(
            rows_v.at[j % 2], out_hbm.at[idx_v.at[j]], sems[j % 2])
    pltpu.sync_copy(gbw_hbm.at[pl.ds(base, _RPW)], grows_v)
    for j in range(_NCHK):
        pltpu.async_copy(grows_v.at[pl.ds(j * _GCH, _GCH)],
                         gbws_hbm.at[idx_v.at[j]], gsem).wait()
    handles[0].wait()
    handles[1].wait()


def _sc_scatter_rows(src, gbw, idx):
    k = functools.partial(
        pl.kernel,
        mesh=plsc.VectorSubcoreMesh(core_axis_name="c", subcore_axis_name="s",
                                    num_cores=_NC, num_subcores=_NS),
        out_type=[
            jax.ShapeDtypeStruct((_N, _D), jnp.float32),
            jax.ShapeDtypeStruct((_N, 128), jnp.float32),
        ],
        scratch_types=[
            pltpu.VMEM((_NCHK, _GCH), jnp.int32),
            pltpu.VMEM((2, _GCH, _D), jnp.float32),
            pltpu.VMEM((_RPW, 128), jnp.float32),
            pltpu.SemaphoreType.DMA,
            pltpu.SemaphoreType.DMA,
            pltpu.SemaphoreType.DMA,
        ],
    )(_sc_scatter_body)
    return k(src, gbw, idx.reshape(_NW, _NCHK, _GCH))


def kernel(x, router_w, router_b, W1, b1, W2, b2, lin_w, lin_b):
    x_flat = x.reshape(_N, _D)
    eid3, gbw, rank3, cnt, aux = _run_router(x_flat, router_w, router_b)

    dst3, offp = _run_dst(eid3, rank3, cnt)
    off = offp[0, :_E + 1]

    xs, gbw_s = _sc_scatter_rows(x_flat, gbw, dst3.reshape(_N))

    ys, logits = _run_ffn(xs, W1, b1, W2, b2, gbw_s, lin_w, lin_b, off)

    moe_flat = _sc_gather_rows(ys, dst3.reshape(_N))
    moe_out = moe_flat.reshape(_B, _S, _D)
    return (logits, moe_out, aux)
